# baseline (device time: 12159 ns/iter reference)
import jax
import jax.numpy as jnp
from jax import lax
from jax.experimental import pallas as pl
from jax.experimental.pallas import tpu as pltpu

N_DEV = 4
K = 8


def kernel(table, idx):
    v_per, d = table.shape
    n = idx.shape[0]
    c = n // K

    def body(table_hbm, idx_hbm, out_hbm, tbl_vmem, idx_vmem, store_buf,
             send_buf, recv_buf, send_sems, recv_sems, store_sems,
             load_sem, idx_sem):
        my_pos = lax.axis_index("i")
        p0 = my_pos ^ 1
        p1 = 3 - my_pos

        barrier_sem = pltpu.get_barrier_semaphore()
        for nbr in [p0, p1]:
            pl.semaphore_signal(
                barrier_sem, inc=1,
                device_id=(nbr,), device_id_type=pl.DeviceIdType.MESH,
            )

        slab = v_per // K
        loads = []
        for k in range(K):
            ld = pltpu.make_async_copy(
                table_hbm.at[pl.ds(k * slab, slab), :],
                tbl_vmem.at[pl.ds(k * slab, slab), :],
                load_sem.at[k],
            )
            ld.start()
            loads.append(ld)
        iload = pltpu.make_async_copy(idx_hbm, idx_vmem, idx_sem)
        iload.start()

        iload.wait()
        lidx = idx_vmem[:, :] - my_pos * v_per
        iota = lax.broadcasted_iota(jnp.int32, (v_per, c), 0)
        ohs = [
            (iota == lidx[:, k * c:(k + 1) * c]).astype(jnp.bfloat16)
            for k in range(K)
        ]
        for ld in loads:
            ld.wait()
        tbl = tbl_vmem[:, :].astype(jnp.bfloat16)

        def partial(k):
            p = lax.dot_general(
                ohs[k], tbl, (((0,), (0,)), ((), ())),
                preferred_element_type=jnp.float32,
            )
            return p.astype(jnp.bfloat16)

        def exchange(slot, partner):
            return pltpu.make_async_remote_copy(
                src_ref=send_buf.at[slot],
                dst_ref=recv_buf.at[slot],
                send_sem=send_sems.at[slot],
                recv_sem=recv_sems.at[slot],
                device_id=(partner,),
                device_id_type=pl.DeviceIdType.MESH,
            )

        order = []
        for j in range(K // 2):
            order += [j, K // 2 + j]
        first = {k: (p0 if k < K // 2 else p1) for k in range(K)}
        second = {k: (p1 if k < K // 2 else p0) for k in range(K)}

        accs = {}
        r0 = {}
        for i, k in enumerate(order):
            acc_k = partial(k)
            send_buf[k, :, :] = acc_k
            if i == 0:
                pl.semaphore_wait(barrier_sem, 2)
            rk = exchange(k, first[k])
            rk.start()
            accs[k] = acc_k
            r0[k] = rk

        r1 = {}
        for k in order:
            r0[k].wait_recv()
            accs[k] = accs[k] + recv_buf[k, :, :]
            send_buf[K + k, :, :] = accs[k]
            rk = exchange(K + k, second[k])
            rk.start()
            r1[k] = rk

        stores = []
        for k in order:
            r1[k].wait_recv()
            store_buf[k, :, :] = accs[k] + recv_buf[K + k, :, :]
            st = pltpu.make_async_copy(
                store_buf.at[k],
                out_hbm.at[pl.ds(k * c, c), :],
                store_sems.at[k],
            )
            st.start()
            stores.append(st)

        for st in stores:
            st.wait()
        for rk in list(r0.values()) + list(r1.values()):
            rk.wait_send()

    call = pl.pallas_call(
        body,
        out_shape=jax.ShapeDtypeStruct((n, d), jnp.bfloat16),
        in_specs=[
            pl.BlockSpec(memory_space=pl.ANY),
            pl.BlockSpec(memory_space=pl.ANY),
        ],
        out_specs=pl.BlockSpec(memory_space=pltpu.MemorySpace.HBM),
        scratch_shapes=[
            pltpu.VMEM((v_per, d), jnp.float32),
            pltpu.VMEM((1, n), jnp.int32),
            pltpu.VMEM((K, c, d), jnp.bfloat16),
            pltpu.VMEM((2 * K, c, d), jnp.bfloat16),
            pltpu.VMEM((2 * K, c, d), jnp.bfloat16),
            pltpu.SemaphoreType.DMA((2 * K,)),
            pltpu.SemaphoreType.DMA((2 * K,)),
            pltpu.SemaphoreType.DMA((K,)),
            pltpu.SemaphoreType.DMA((K,)),
            pltpu.SemaphoreType.DMA,
        ],
        compiler_params=pltpu.CompilerParams(collective_id=0),
    )
    table = pltpu.with_memory_space_constraint(table, pltpu.MemorySpace.HBM)
    idx2 = pltpu.with_memory_space_constraint(
        idx.reshape(1, n), pltpu.MemorySpace.HBM
    )
    return call(table, idx2)


# device time: 11415 ns/iter; 1.0652x vs baseline; 1.0652x over previous
import jax
import jax.numpy as jnp
from jax import lax
from jax.experimental import pallas as pl
from jax.experimental.pallas import tpu as pltpu

N_DEV = 4
K = 4


def kernel(table, idx):
    v_per, d = table.shape
    n = idx.shape[0]
    c = n // K

    def body(table_hbm, idx_hbm, out_hbm, tbl_vmem, idx_vmem, store_buf,
             send_buf, recv_buf, send_sems, recv_sems, store_sems,
             load_sem, idx_sem):
        my_pos = lax.axis_index("i")
        p0 = my_pos ^ 1
        p1 = 3 - my_pos

        barrier_sem = pltpu.get_barrier_semaphore()
        for nbr in [p0, p1]:
            pl.semaphore_signal(
                barrier_sem, inc=1,
                device_id=(nbr,), device_id_type=pl.DeviceIdType.MESH,
            )

        slab = v_per // K
        loads = []
        for k in range(K):
            ld = pltpu.make_async_copy(
                table_hbm.at[pl.ds(k * slab, slab), :],
                tbl_vmem.at[pl.ds(k * slab, slab), :],
                load_sem.at[k],
            )
            ld.start()
            loads.append(ld)
        iload = pltpu.make_async_copy(idx_hbm, idx_vmem, idx_sem)
        iload.start()

        iload.wait()
        lidx = idx_vmem[:, :] - my_pos * v_per
        iota = lax.broadcasted_iota(jnp.int32, (v_per, c), 0)
        ohs = [
            (iota == lidx[:, k * c:(k + 1) * c]).astype(jnp.bfloat16)
            for k in range(K)
        ]
        for ld in loads:
            ld.wait()
        tbl = tbl_vmem[:, :].astype(jnp.bfloat16)

        def partial(k):
            p = lax.dot_general(
                ohs[k], tbl, (((0,), (0,)), ((), ())),
                preferred_element_type=jnp.float32,
            )
            return p.astype(jnp.bfloat16)

        def exchange(slot, partner):
            return pltpu.make_async_remote_copy(
                src_ref=send_buf.at[slot],
                dst_ref=recv_buf.at[slot],
                send_sem=send_sems.at[slot],
                recv_sem=recv_sems.at[slot],
                device_id=(partner,),
                device_id_type=pl.DeviceIdType.MESH,
            )

        order = []
        for j in range(K // 2):
            order += [j, K // 2 + j]
        first = {k: (p0 if k < K // 2 else p1) for k in range(K)}
        second = {k: (p1 if k < K // 2 else p0) for k in range(K)}

        accs = {}
        r0 = {}
        for i, k in enumerate(order):
            acc_k = partial(k)
            send_buf[k, :, :] = acc_k
            if i == 0:
                pl.semaphore_wait(barrier_sem, 2)
            rk = exchange(k, first[k])
            rk.start()
            accs[k] = acc_k
            r0[k] = rk

        r1 = {}
        for k in order:
            r0[k].wait_recv()
            accs[k] = accs[k] + recv_buf[k, :, :]
            send_buf[K + k, :, :] = accs[k]
            rk = exchange(K + k, second[k])
            rk.start()
            r1[k] = rk

        stores = []
        for k in order:
            r1[k].wait_recv()
            store_buf[k, :, :] = accs[k] + recv_buf[K + k, :, :]
            st = pltpu.make_async_copy(
                store_buf.at[k],
                out_hbm.at[pl.ds(k * c, c), :],
                store_sems.at[k],
            )
            st.start()
            stores.append(st)

        for st in stores:
            st.wait()
        for rk in list(r0.values()) + list(r1.values()):
            rk.wait_send()

    call = pl.pallas_call(
        body,
        out_shape=jax.ShapeDtypeStruct((n, d), jnp.bfloat16),
        in_specs=[
            pl.BlockSpec(memory_space=pl.ANY),
            pl.BlockSpec(memory_space=pl.ANY),
        ],
        out_specs=pl.BlockSpec(memory_space=pltpu.MemorySpace.HBM),
        scratch_shapes=[
            pltpu.VMEM((v_per, d), jnp.float32),
            pltpu.VMEM((1, n), jnp.int32),
            pltpu.VMEM((K, c, d), jnp.bfloat16),
            pltpu.VMEM((2 * K, c, d), jnp.bfloat16),
            pltpu.VMEM((2 * K, c, d), jnp.bfloat16),
            pltpu.SemaphoreType.DMA((2 * K,)),
            pltpu.SemaphoreType.DMA((2 * K,)),
            pltpu.SemaphoreType.DMA((K,)),
            pltpu.SemaphoreType.DMA((K,)),
            pltpu.SemaphoreType.DMA,
        ],
        compiler_params=pltpu.CompilerParams(collective_id=0),
    )
    table = pltpu.with_memory_space_constraint(table, pltpu.MemorySpace.HBM)
    idx2 = pltpu.with_memory_space_constraint(
        idx.reshape(1, n), pltpu.MemorySpace.HBM
    )
    return call(table, idx2)
